# Initial kernel scaffold; baseline (speedup 1.0000x reference)
import jax
import jax.numpy as jnp
from jax.experimental import pallas as pl


def kernel(loc_data, conf_data, prior_data):
    raise NotImplementedError("write your pallas kernel here")


# speedup vs baseline: 240.0641x; 240.0641x over previous
import functools

import jax
import jax.numpy as jnp
from jax import lax
from jax.experimental import pallas as pl
from jax.experimental.pallas import tpu as pltpu
from jax.experimental.pallas import tpu_sc as plsc

_TOP_K = 200
_CONF = 0.01
_NMS = 0.45
_VAR0 = 0.1
_VAR1 = 0.2
_P = 5120
_PART = 640
_NV = _PART // 16
_NEG = -1e30
_BIGF = float(2 ** 30)


def _nms_body(s_hbm, x1_hbm, y1_hbm, x2_hbm, y2_hbm, out_hbm, cand_hbm,
              sref, x1r, y1r, x2r, y2r, ar, rowb, candb, outb):
    c = lax.axis_index("c")
    sid = lax.axis_index("s")
    b = c * 2 + sid // 8
    p = sid % 8
    off = p * _PART

    pltpu.sync_copy(s_hbm.at[b, pl.ds(off, _PART)], sref)
    pltpu.sync_copy(x1_hbm.at[b, pl.ds(off, _PART)], x1r)
    pltpu.sync_copy(y1_hbm.at[b, pl.ds(off, _PART)], y1r)
    pltpu.sync_copy(x2_hbm.at[b, pl.ds(off, _PART)], x2r)
    pltpu.sync_copy(y2_hbm.at[b, pl.ds(off, _PART)], y2r)

    iota_i = lax.iota(jnp.int32, 16)
    e0, e1, e2, e3, e4, e5, e6 = (
        jnp.where(iota_i == i, 1.0, 0.0) for i in range(7))

    def init_body(v, carry):
        idx = iota_i + v * 16
        x1v = plsc.load_gather(x1r, [idx])
        y1v = plsc.load_gather(y1r, [idx])
        x2v = plsc.load_gather(x2r, [idx])
        y2v = plsc.load_gather(y2r, [idx])
        plsc.store_scatter(ar, [idx], (x2v - x1v) * (y2v - y1v))
        sv = plsc.load_gather(sref, [idx])
        plsc.store_scatter(sref, [idx], jnp.where(sv > _CONF, sv, _NEG))
        return carry

    lax.fori_loop(0, _NV, init_body, 0)

    def body(k, carry):
        def am_body(v, carry):
            bk, bv = carry
            idx = iota_i + v * 16
            sv = plsc.load_gather(sref, [idx])
            gv = (idx + off).astype(jnp.float32)
            upd = sv > bk
            return jnp.where(upd, sv, bk), jnp.where(upd, gv, bv)

        bk, bv = lax.fori_loop(0, _NV, am_body, (
            jnp.full((16,), _NEG, jnp.float32),
            jnp.full((16,), _BIGF, jnp.float32)))
        mt = jnp.max(bk)
        vt = jnp.min(jnp.where(bk == mt, bv, _BIGF))
        li = jnp.where(mt > _CONF, vt.astype(jnp.int32) - off, 0)
        liv = jnp.full((16,), li, jnp.int32)
        gx1 = plsc.load_gather(x1r, [liv])
        gy1 = plsc.load_gather(y1r, [liv])
        gx2 = plsc.load_gather(x2r, [liv])
        gy2 = plsc.load_gather(y2r, [liv])
        gar = plsc.load_gather(ar, [liv])
        row = (jnp.full((16,), mt, jnp.float32) * e0 + gx1 * e1 + gy1 * e2
               + gx2 * e3 + gy2 * e4 + gar * e5
               + jnp.full((16,), vt, jnp.float32) * e6)
        rowb[...] = row

        pltpu.sync_copy(rowb, cand_hbm.at[b * 8 + p])
        plsc.subcore_barrier()
        pltpu.sync_copy(cand_hbm.at[pl.ds(b * 8, 8)], candb)
        plsc.subcore_barrier()

        r8 = jnp.minimum(iota_i, 7)
        s8 = plsc.load_gather(candb, [r8, jnp.full((16,), 0, jnp.int32)])
        g8 = plsc.load_gather(candb, [r8, jnp.full((16,), 6, jnp.int32)])
        m = jnp.max(s8)
        found = m > _CONF
        widx = jnp.min(jnp.where(s8 == m, g8, _BIGF))
        jstar = jnp.where(found, widx.astype(jnp.int32) // _PART, 0)
        jv = jnp.full((16,), jstar, jnp.int32)
        wx1 = plsc.load_gather(candb, [jv, jnp.full((16,), 1, jnp.int32)])
        wy1 = plsc.load_gather(candb, [jv, jnp.full((16,), 2, jnp.int32)])
        wx2 = plsc.load_gather(candb, [jv, jnp.full((16,), 3, jnp.int32)])
        wy2 = plsc.load_gather(candb, [jv, jnp.full((16,), 4, jnp.int32)])
        war = plsc.load_gather(candb, [jv, jnp.full((16,), 5, jnp.int32)])


        @pl.when(p == 0)
        def _():
            scale = jnp.full((16,), jnp.where(found, 1.0, 0.0), jnp.float32)
            orow = (jnp.full((16,), m, jnp.float32) * e0 + wx1 * e1
                    + wy1 * e2 + wx2 * e3 + wy2 * e4) * scale
            plsc.store_scatter(
                outb, [jnp.full((16,), k, jnp.int32), iota_i], orow)

        thrv = jnp.full((16,), jnp.where(found, _NMS, 2e30), jnp.float32)

        def sup_body(v, carry):
            idx = iota_i + v * 16
            x1v = plsc.load_gather(x1r, [idx])
            y1v = plsc.load_gather(y1r, [idx])
            x2v = plsc.load_gather(x2r, [idx])
            y2v = plsc.load_gather(y2r, [idx])
            av = plsc.load_gather(ar, [idx])
            xx1 = jnp.maximum(wx1, x1v)
            yy1 = jnp.maximum(wy1, y1v)
            xx2 = jnp.minimum(wx2, x2v)
            yy2 = jnp.minimum(wy2, y2v)
            inter = jnp.maximum(xx2 - xx1, 0.0) * jnp.maximum(yy2 - yy1, 0.0)
            union = (war + av) - inter
            iou = inter / jnp.maximum(union, 1e-12)
            kill = iou > thrv
            sv = plsc.load_gather(sref, [idx])
            plsc.store_scatter(sref, [idx], jnp.where(kill, _NEG, sv))
            return carry

        lax.fori_loop(0, _NV, sup_body, 0)
        return carry

    lax.fori_loop(0, _TOP_K, body, 0)

    @pl.when(p == 0)
    def _():
        pltpu.sync_copy(outb, out_hbm.at[b])


def _run_nms(s_p, x1_p, y1_p, x2_p, y2_p):
    num = s_p.shape[0]
    mesh = plsc.VectorSubcoreMesh(core_axis_name="c", subcore_axis_name="s",
                                  num_cores=2, num_subcores=16)
    f = pl.kernel(
        _nms_body,
        out_type=(jax.ShapeDtypeStruct((num, _TOP_K, 16), jnp.float32),
                  jax.ShapeDtypeStruct((32, 16), jnp.float32)),
        mesh=mesh,
        compiler_params=pltpu.CompilerParams(needs_layout_passes=False),
        scratch_types=[
            pltpu.VMEM((_PART,), jnp.float32),
            pltpu.VMEM((_PART,), jnp.float32),
            pltpu.VMEM((_PART,), jnp.float32),
            pltpu.VMEM((_PART,), jnp.float32),
            pltpu.VMEM((_PART,), jnp.float32),
            pltpu.VMEM((_PART,), jnp.float32),
            pltpu.VMEM((16,), jnp.float32),
            pltpu.VMEM((8, 16), jnp.float32),
            pltpu.VMEM((_TOP_K, 16), jnp.float32),
        ],
    )
    return f(s_p, x1_p, y1_p, x2_p, y2_p)[0]


def kernel(loc_data, conf_data, prior_data):
    num = loc_data.shape[0]
    conf = jax.nn.softmax(conf_data, axis=-1)
    scores = conf[:, :, 1]
    decs = []
    for i in range(num):
        loci = loc_data[i]
        xy = prior_data[:, :2] + loci[:, :2] * _VAR0 * prior_data[:, 2:]
        wh = prior_data[:, 2:] * jnp.exp(loci[:, 2:] * _VAR1)
        x1y1 = xy - wh / 2.0
        x2y2 = wh + x1y1
        decs.append(jnp.concatenate([x1y1, x2y2], axis=1))
    dec = jnp.stack(decs)
    pad = _P - scores.shape[1]

    def padp(a):
        return jnp.pad(a, ((0, 0), (0, pad)))

    raw = _run_nms(padp(scores),
                   padp(dec[:, :, 0]), padp(dec[:, :, 1]),
                   padp(dec[:, :, 2]), padp(dec[:, :, 3]))
    out = jnp.zeros((num, 2, _TOP_K, 5), jnp.float32)
    return out.at[:, 1].set(raw[:, :_TOP_K, :5])


# speedup vs baseline: 273.1319x; 1.1377x over previous
import functools

import jax
import jax.numpy as jnp
from jax import lax
from jax.experimental import pallas as pl
from jax.experimental.pallas import tpu as pltpu
from jax.experimental.pallas import tpu_sc as plsc

_TOP_K = 200
_CONF = 0.01
_NMS = 0.45
_VAR0 = 0.1
_VAR1 = 0.2
_P = 5120
_PART = 640
_NV = _PART // 16
_NEG = -1e30
_BIGF = float(2 ** 30)


def _nms_body(s_hbm, x1_hbm, y1_hbm, x2_hbm, y2_hbm, out_hbm, cand_hbm,
              sref, x1r, y1r, x2r, y2r, ar, rowb, candb, outb):
    c = lax.axis_index("c")
    sid = lax.axis_index("s")
    b = c * 2 + sid // 8
    p = sid % 8
    off = p * _PART

    pltpu.sync_copy(s_hbm.at[b, pl.ds(off, _PART)], sref)
    pltpu.sync_copy(x1_hbm.at[b, pl.ds(off, _PART)], x1r)
    pltpu.sync_copy(y1_hbm.at[b, pl.ds(off, _PART)], y1r)
    pltpu.sync_copy(x2_hbm.at[b, pl.ds(off, _PART)], x2r)
    pltpu.sync_copy(y2_hbm.at[b, pl.ds(off, _PART)], y2r)

    iota_i = lax.iota(jnp.int32, 16)
    e0, e1, e2, e3, e4, e5, e6 = (
        jnp.where(iota_i == i, 1.0, 0.0) for i in range(7))

    def init_body(v, carry):
        bk, bv = carry
        idx = iota_i + v * 16
        x1v = plsc.load_gather(x1r, [idx])
        y1v = plsc.load_gather(y1r, [idx])
        x2v = plsc.load_gather(x2r, [idx])
        y2v = plsc.load_gather(y2r, [idx])
        plsc.store_scatter(ar, [idx], (x2v - x1v) * (y2v - y1v))
        sv = plsc.load_gather(sref, [idx])
        nsv = jnp.where(sv > _CONF, sv, _NEG)
        plsc.store_scatter(sref, [idx], nsv)
        upd = nsv > bk
        return (jnp.where(upd, nsv, bk),
                jnp.where(upd, (idx + off).astype(jnp.float32), bv))

    bk0, bv0 = lax.fori_loop(0, _NV, init_body, (
        jnp.full((16,), _NEG, jnp.float32),
        jnp.full((16,), _BIGF, jnp.float32)))

    def body(k, carry):
        bk, bv = carry
        mt = jnp.max(bk)
        vt = jnp.min(jnp.where(bk == mt, bv, _BIGF))
        li = jnp.where(mt > _CONF, vt.astype(jnp.int32) - off, 0)
        liv = jnp.full((16,), li, jnp.int32)
        gx1 = plsc.load_gather(x1r, [liv])
        gy1 = plsc.load_gather(y1r, [liv])
        gx2 = plsc.load_gather(x2r, [liv])
        gy2 = plsc.load_gather(y2r, [liv])
        gar = plsc.load_gather(ar, [liv])
        row = (jnp.full((16,), mt, jnp.float32) * e0 + gx1 * e1 + gy1 * e2
               + gx2 * e3 + gy2 * e4 + gar * e5
               + jnp.full((16,), vt, jnp.float32) * e6)
        rowb[...] = row

        par = k % 2
        pltpu.sync_copy(rowb, cand_hbm.at[par, b * 8 + p])
        plsc.subcore_barrier()
        pltpu.sync_copy(cand_hbm.at[par, pl.ds(b * 8, 8)], candb)

        r8 = jnp.minimum(iota_i, 7)
        s8 = plsc.load_gather(candb, [r8, jnp.full((16,), 0, jnp.int32)])
        g8 = plsc.load_gather(candb, [r8, jnp.full((16,), 6, jnp.int32)])
        m = jnp.max(s8)
        found = m > _CONF
        widx = jnp.min(jnp.where(s8 == m, g8, _BIGF))
        jstar = jnp.where(found, widx.astype(jnp.int32) // _PART, 0)
        jv = jnp.full((16,), jstar, jnp.int32)
        wx1 = plsc.load_gather(candb, [jv, jnp.full((16,), 1, jnp.int32)])
        wy1 = plsc.load_gather(candb, [jv, jnp.full((16,), 2, jnp.int32)])
        wx2 = plsc.load_gather(candb, [jv, jnp.full((16,), 3, jnp.int32)])
        wy2 = plsc.load_gather(candb, [jv, jnp.full((16,), 4, jnp.int32)])
        war = plsc.load_gather(candb, [jv, jnp.full((16,), 5, jnp.int32)])


        @pl.when(p == 0)
        def _():
            scale = jnp.full((16,), jnp.where(found, 1.0, 0.0), jnp.float32)
            orow = (jnp.full((16,), m, jnp.float32) * e0 + wx1 * e1
                    + wy1 * e2 + wx2 * e3 + wy2 * e4) * scale
            plsc.store_scatter(
                outb, [jnp.full((16,), k, jnp.int32), iota_i], orow)

        thrv = jnp.full((16,), jnp.where(found, _NMS, 2e30), jnp.float32)

        def sup_body(v, carry):
            nbk, nbv = carry
            idx = iota_i + v * 16
            x1v = plsc.load_gather(x1r, [idx])
            y1v = plsc.load_gather(y1r, [idx])
            x2v = plsc.load_gather(x2r, [idx])
            y2v = plsc.load_gather(y2r, [idx])
            av = plsc.load_gather(ar, [idx])
            xx1 = jnp.maximum(wx1, x1v)
            yy1 = jnp.maximum(wy1, y1v)
            xx2 = jnp.minimum(wx2, x2v)
            yy2 = jnp.minimum(wy2, y2v)
            inter = jnp.maximum(xx2 - xx1, 0.0) * jnp.maximum(yy2 - yy1, 0.0)
            union = (war + av) - inter
            iou = inter / jnp.maximum(union, 1e-12)
            kill = iou > thrv
            sv = plsc.load_gather(sref, [idx])
            nsv = jnp.where(kill, _NEG, sv)
            plsc.store_scatter(sref, [idx], nsv)
            upd = nsv > nbk
            return (jnp.where(upd, nsv, nbk),
                    jnp.where(upd, (idx + off).astype(jnp.float32), nbv))

        return lax.fori_loop(0, _NV, sup_body, (
            jnp.full((16,), _NEG, jnp.float32),
            jnp.full((16,), _BIGF, jnp.float32)))

    lax.fori_loop(0, _TOP_K, body, (bk0, bv0))

    @pl.when(p == 0)
    def _():
        pltpu.sync_copy(outb, out_hbm.at[b])


def _run_nms(s_p, x1_p, y1_p, x2_p, y2_p):
    num = s_p.shape[0]
    mesh = plsc.VectorSubcoreMesh(core_axis_name="c", subcore_axis_name="s",
                                  num_cores=2, num_subcores=16)
    f = pl.kernel(
        _nms_body,
        out_type=(jax.ShapeDtypeStruct((num, _TOP_K, 16), jnp.float32),
                  jax.ShapeDtypeStruct((2, 32, 16), jnp.float32)),
        mesh=mesh,
        compiler_params=pltpu.CompilerParams(needs_layout_passes=False),
        scratch_types=[
            pltpu.VMEM((_PART,), jnp.float32),
            pltpu.VMEM((_PART,), jnp.float32),
            pltpu.VMEM((_PART,), jnp.float32),
            pltpu.VMEM((_PART,), jnp.float32),
            pltpu.VMEM((_PART,), jnp.float32),
            pltpu.VMEM((_PART,), jnp.float32),
            pltpu.VMEM((16,), jnp.float32),
            pltpu.VMEM((8, 16), jnp.float32),
            pltpu.VMEM((_TOP_K, 16), jnp.float32),
        ],
    )
    return f(s_p, x1_p, y1_p, x2_p, y2_p)[0]


def kernel(loc_data, conf_data, prior_data):
    num = loc_data.shape[0]
    conf = jax.nn.softmax(conf_data, axis=-1)
    scores = conf[:, :, 1]
    decs = []
    for i in range(num):
        loci = loc_data[i]
        xy = prior_data[:, :2] + loci[:, :2] * _VAR0 * prior_data[:, 2:]
        wh = prior_data[:, 2:] * jnp.exp(loci[:, 2:] * _VAR1)
        x1y1 = xy - wh / 2.0
        x2y2 = wh + x1y1
        decs.append(jnp.concatenate([x1y1, x2y2], axis=1))
    dec = jnp.stack(decs)
    pad = _P - scores.shape[1]

    def padp(a):
        return jnp.pad(a, ((0, 0), (0, pad)))

    raw = _run_nms(padp(scores),
                   padp(dec[:, :, 0]), padp(dec[:, :, 1]),
                   padp(dec[:, :, 2]), padp(dec[:, :, 3]))
    out = jnp.zeros((num, 2, _TOP_K, 5), jnp.float32)
    return out.at[:, 1].set(raw[:, :_TOP_K, :5])


# speedup vs baseline: 371.7190x; 1.3610x over previous
import functools

import jax
import jax.numpy as jnp
from jax import lax
from jax.experimental import pallas as pl
from jax.experimental.pallas import tpu as pltpu
from jax.experimental.pallas import tpu_sc as plsc

_TOP_K = 200
_CONF = 0.01
_NMS = 0.45
_VAR0 = 0.1
_VAR1 = 0.2
_P = 5120
_PART = 640
_NV = _PART // 16
_NEG = -1e30
_BIGF = float(2 ** 30)


def _nms_body(s_hbm, x1_hbm, y1_hbm, x2_hbm, y2_hbm, out_hbm, cand_hbm,
              sref, x1r, y1r, x2r, y2r, ar, rowb, candb, outb):
    c = lax.axis_index("c")
    sid = lax.axis_index("s")
    b = c * 2 + sid // 8
    p = sid % 8
    off = p * _PART

    pltpu.sync_copy(s_hbm.at[b, pl.ds(off, _PART)], sref)
    pltpu.sync_copy(x1_hbm.at[b, pl.ds(off, _PART)], x1r)
    pltpu.sync_copy(y1_hbm.at[b, pl.ds(off, _PART)], y1r)
    pltpu.sync_copy(x2_hbm.at[b, pl.ds(off, _PART)], x2r)
    pltpu.sync_copy(y2_hbm.at[b, pl.ds(off, _PART)], y2r)

    iota_i = lax.iota(jnp.int32, 16)
    e0, e1, e2, e3, e4, e5, e6 = (
        jnp.where(iota_i == i, 1.0, 0.0) for i in range(7))

    def init_body(v, carry):
        bk, bv = carry
        idx = iota_i + v * 16
        x1v = plsc.load_gather(x1r, [idx])
        y1v = plsc.load_gather(y1r, [idx])
        x2v = plsc.load_gather(x2r, [idx])
        y2v = plsc.load_gather(y2r, [idx])
        plsc.store_scatter(ar, [idx], (x2v - x1v) * (y2v - y1v))
        sv = plsc.load_gather(sref, [idx])
        nsv = jnp.where(sv > _CONF, sv, _NEG)
        plsc.store_scatter(sref, [idx], nsv)
        upd = nsv > bk
        return (jnp.where(upd, nsv, bk),
                jnp.where(upd, (idx + off).astype(jnp.float32), bv))

    bk0, bv0 = lax.fori_loop(0, _NV, init_body, (
        jnp.full((16,), _NEG, jnp.float32),
        jnp.full((16,), _BIGF, jnp.float32)))

    def body(k, carry):
        bk, bv = carry
        mt = jnp.max(bk)
        vt = jnp.min(jnp.where(bk == mt, bv, _BIGF))
        li = jnp.where(mt > _CONF, vt.astype(jnp.int32) - off, 0)
        liv = jnp.full((16,), li, jnp.int32)
        gx1 = plsc.load_gather(x1r, [liv])
        gy1 = plsc.load_gather(y1r, [liv])
        gx2 = plsc.load_gather(x2r, [liv])
        gy2 = plsc.load_gather(y2r, [liv])
        gar = plsc.load_gather(ar, [liv])
        row = (jnp.full((16,), mt, jnp.float32) * e0 + gx1 * e1 + gy1 * e2
               + gx2 * e3 + gy2 * e4 + gar * e5
               + jnp.full((16,), vt, jnp.float32) * e6)
        rowb[...] = row

        par = k % 2
        pltpu.sync_copy(rowb, cand_hbm.at[par, b * 8 + p])
        plsc.subcore_barrier()
        pltpu.sync_copy(cand_hbm.at[par, pl.ds(b * 8, 8)], candb)

        r8 = jnp.minimum(iota_i, 7)
        s8 = plsc.load_gather(candb, [r8, jnp.full((16,), 0, jnp.int32)])
        g8 = plsc.load_gather(candb, [r8, jnp.full((16,), 6, jnp.int32)])
        m = jnp.max(s8)
        found = m > _CONF
        widx = jnp.min(jnp.where(s8 == m, g8, _BIGF))
        jstar = jnp.where(found, widx.astype(jnp.int32) // _PART, 0)
        jv = jnp.full((16,), jstar, jnp.int32)
        wx1 = plsc.load_gather(candb, [jv, jnp.full((16,), 1, jnp.int32)])
        wy1 = plsc.load_gather(candb, [jv, jnp.full((16,), 2, jnp.int32)])
        wx2 = plsc.load_gather(candb, [jv, jnp.full((16,), 3, jnp.int32)])
        wy2 = plsc.load_gather(candb, [jv, jnp.full((16,), 4, jnp.int32)])
        war = plsc.load_gather(candb, [jv, jnp.full((16,), 5, jnp.int32)])


        @pl.when(p == 0)
        def _():
            scale = jnp.full((16,), jnp.where(found, 1.0, 0.0), jnp.float32)
            orow = (jnp.full((16,), m, jnp.float32) * e0 + wx1 * e1
                    + wy1 * e2 + wx2 * e3 + wy2 * e4) * scale
            plsc.store_scatter(
                outb, [jnp.full((16,), k, jnp.int32), iota_i], orow)

        thrv = jnp.full((16,), jnp.where(found, _NMS, 2e30), jnp.float32)

        @plsc.parallel_loop(0, _NV, unroll=4, carry=(
            jnp.full((16,), _NEG, jnp.float32),
            jnp.full((16,), _BIGF, jnp.float32)))
        def sup_scan(v, carry):
            nbk, nbv = carry
            idx = iota_i + v * 16
            x1v = plsc.load_gather(x1r, [idx])
            y1v = plsc.load_gather(y1r, [idx])
            x2v = plsc.load_gather(x2r, [idx])
            y2v = plsc.load_gather(y2r, [idx])
            av = plsc.load_gather(ar, [idx])
            xx1 = jnp.maximum(wx1, x1v)
            yy1 = jnp.maximum(wy1, y1v)
            xx2 = jnp.minimum(wx2, x2v)
            yy2 = jnp.minimum(wy2, y2v)
            inter = jnp.maximum(xx2 - xx1, 0.0) * jnp.maximum(yy2 - yy1, 0.0)
            union = (war + av) - inter
            iou = inter / jnp.maximum(union, 1e-12)
            kill = iou > thrv
            sv = plsc.load_gather(sref, [idx])
            nsv = jnp.where(kill, _NEG, sv)
            plsc.store_scatter(sref, [idx], nsv)
            gv = (idx + off).astype(jnp.float32)
            upd = (nsv > nbk) | ((nsv == nbk) & (gv < nbv))
            return (jnp.where(upd, nsv, nbk), jnp.where(upd, gv, nbv))

        return sup_scan

    lax.fori_loop(0, _TOP_K, body, (bk0, bv0))

    @pl.when(p == 0)
    def _():
        pltpu.sync_copy(outb, out_hbm.at[b])


def _run_nms(s_p, x1_p, y1_p, x2_p, y2_p):
    num = s_p.shape[0]
    mesh = plsc.VectorSubcoreMesh(core_axis_name="c", subcore_axis_name="s",
                                  num_cores=2, num_subcores=16)
    f = pl.kernel(
        _nms_body,
        out_type=(jax.ShapeDtypeStruct((num, _TOP_K, 16), jnp.float32),
                  jax.ShapeDtypeStruct((2, 32, 16), jnp.float32)),
        mesh=mesh,
        compiler_params=pltpu.CompilerParams(needs_layout_passes=False),
        scratch_types=[
            pltpu.VMEM((_PART,), jnp.float32),
            pltpu.VMEM((_PART,), jnp.float32),
            pltpu.VMEM((_PART,), jnp.float32),
            pltpu.VMEM((_PART,), jnp.float32),
            pltpu.VMEM((_PART,), jnp.float32),
            pltpu.VMEM((_PART,), jnp.float32),
            pltpu.VMEM((16,), jnp.float32),
            pltpu.VMEM((8, 16), jnp.float32),
            pltpu.VMEM((_TOP_K, 16), jnp.float32),
        ],
    )
    return f(s_p, x1_p, y1_p, x2_p, y2_p)[0]


def kernel(loc_data, conf_data, prior_data):
    num = loc_data.shape[0]
    conf = jax.nn.softmax(conf_data, axis=-1)
    scores = conf[:, :, 1]
    decs = []
    for i in range(num):
        loci = loc_data[i]
        xy = prior_data[:, :2] + loci[:, :2] * _VAR0 * prior_data[:, 2:]
        wh = prior_data[:, 2:] * jnp.exp(loci[:, 2:] * _VAR1)
        x1y1 = xy - wh / 2.0
        x2y2 = wh + x1y1
        decs.append(jnp.concatenate([x1y1, x2y2], axis=1))
    dec = jnp.stack(decs)
    pad = _P - scores.shape[1]

    def padp(a):
        return jnp.pad(a, ((0, 0), (0, pad)))

    raw = _run_nms(padp(scores),
                   padp(dec[:, :, 0]), padp(dec[:, :, 1]),
                   padp(dec[:, :, 2]), padp(dec[:, :, 3]))
    out = jnp.zeros((num, 2, _TOP_K, 5), jnp.float32)
    return out.at[:, 1].set(raw[:, :_TOP_K, :5])
